# baseline (speedup 1.0000x reference)
import functools

import jax
import jax.numpy as jnp
from jax import lax
from jax.experimental import pallas as pl
from jax.experimental.pallas import tpu as pltpu
from jax.experimental.pallas import tpu_sc as plsc

_STRIDE = 8.0
_NC, _NS, _L = 2, 16, 16
_NW = _NC * _NS
_CHUNK = 512


def _sigmoid16(x):
    return 1.0 / (1.0 + jnp.exp(-x))


def _sqrt16(y):
    i = lax.bitcast_convert_type(y, jnp.int32)
    x = lax.bitcast_convert_type((i >> 1) + 0x1FBD1DF5, jnp.float32)
    x = 0.5 * (x + y / x)
    x = 0.5 * (x + y / x)
    return x


def _sc_body(center_hbm, logits_hbm, idxo_hbm, scoreo_hbm,
             lbuf, cbuf, oibuf, osbuf, lsem, *, n_pix, n_cls):
    ppw = n_pix // _NW
    nch = ppw // _CHUNK
    cw = _CHUNK * n_cls
    wid = lax.axis_index("s") * _NC + lax.axis_index("c")
    base = wid * ppw

    def logits_copy(ch, slot):
        return pltpu.make_async_copy(
            logits_hbm.at[pl.ds((base + ch * _CHUNK) * n_cls, cw)],
            lbuf.at[pl.ds(slot * cw, cw)], lsem.at[slot])

    pltpu.sync_copy(center_hbm.at[pl.ds(base, ppw)], cbuf)
    logits_copy(0, 0).start()

    @pl.loop(0, nch)
    def _chunk(ch):
        slot = lax.rem(ch, 2)
        logits_copy(ch, slot).wait()

        @pl.when(ch + 1 < nch)
        def _():
            logits_copy(ch + 1, 1 - slot).start()

        @pl.loop(0, _CHUNK // _L)
        def _group(g):
            p0 = ch * _CHUNK + g * _L
            lane = lax.broadcasted_iota(jnp.int32, (_L,), 0)
            rowbase = slot * cw + (g * _L + lane) * n_cls
            bests, bidxs = [], []
            for k in range(4):
                bests.append(plsc.load_gather(lbuf, [rowbase + k]))
                bidxs.append(jnp.full((_L,), k, jnp.int32))
            for c in range(4, 20):
                k = c & 3
                v = plsc.load_gather(lbuf, [rowbase + c])
                gt = v > bests[k]
                bidxs[k] = jnp.where(gt, c, bidxs[k])
                bests[k] = jnp.where(gt, v, bests[k])
            best, bidx = bests[0], bidxs[0]
            for k in range(1, 4):
                gt = bests[k] > best
                bidx = jnp.where(gt, bidxs[k], bidx)
                best = jnp.where(gt, bests[k], best)
            oibuf[pl.ds(p0, _L)] = bidx
            cen = cbuf[pl.ds(p0, _L)]
            osbuf[pl.ds(p0, _L)] = _sqrt16(_sigmoid16(cen) * _sigmoid16(best))

    pltpu.sync_copy(oibuf, idxo_hbm.at[pl.ds(base, ppw)])
    pltpu.sync_copy(osbuf, scoreo_hbm.at[pl.ds(base, ppw)])


def _tc_bbox_body(bbox_ref, out_ref, *, n_h):
    e = jnp.exp(bbox_ref[...]) * _STRIDE
    lane = lax.broadcasted_iota(jnp.int32, e.shape, 1)
    lo_half = (lane & 2) == 0
    partner = jnp.where(lo_half, jnp.roll(e, -2, axis=1),
                        jnp.roll(e, 2, axis=1))
    rvec = (pl.program_id(0) * e.shape[0]
            + lax.broadcasted_iota(jnp.int32, e.shape, 0))
    wx = ((lax.rem(rvec, 4) * 32 + (lane >> 2)).astype(jnp.float32)
          * _STRIDE + _STRIDE * 0.5)
    hy = (lax.rem(rvec >> 2, n_h).astype(jnp.float32)
          * _STRIDE + _STRIDE * 0.5)
    base = jnp.where((lane & 3) == 0, wx, hy)
    out_ref[...] = jnp.where(lo_half, base + (partner - e) * 0.5,
                             partner + e)


def kernel(bbox, center, class_logits, img_h, img_w):
    nB, nH, nW, nCls = class_logits.shape
    n = nB * nH * nW

    center1 = center.reshape(n)
    logits1 = class_logits.reshape(n * nCls)

    mesh = plsc.VectorSubcoreMesh(core_axis_name="c", subcore_axis_name="s")
    sc = pl.kernel(
        functools.partial(_sc_body, n_pix=n, n_cls=nCls),
        mesh=mesh,
        out_type=[
            jax.ShapeDtypeStruct((n,), jnp.int32),
            jax.ShapeDtypeStruct((n,), jnp.float32),
        ],
        scratch_types=[
            pltpu.VMEM((2 * _CHUNK * nCls,), jnp.float32),
            pltpu.VMEM((n // _NW,), jnp.float32),
            pltpu.VMEM((n // _NW,), jnp.int32),
            pltpu.VMEM((n // _NW,), jnp.float32),
            pltpu.SemaphoreType.DMA((2,)),
        ],
        compiler_params=pltpu.CompilerParams(needs_layout_passes=False),
    )
    idx, score = sc(center1, logits1)

    rows = 512
    bbox2 = bbox.reshape(n // 32, 128)
    bbox_out = pl.pallas_call(
        functools.partial(_tc_bbox_body, n_h=nH),
        grid=(n // 32 // rows,),
        in_specs=[pl.BlockSpec((rows, 128), lambda i: (i, 0))],
        out_specs=pl.BlockSpec((rows, 128), lambda i: (i, 0)),
        out_shape=jax.ShapeDtypeStruct((n // 32, 128), jnp.float32),
    )(bbox2)

    return (bbox_out.reshape(nB, nH * nW, 4),
            idx.reshape(nB, nH * nW),
            score.reshape(nB, nH * nW))


# speedup vs baseline: 1.0099x; 1.0099x over previous
import functools

import jax
import jax.numpy as jnp
from jax import lax
from jax.experimental import pallas as pl
from jax.experimental.pallas import tpu as pltpu
from jax.experimental.pallas import tpu_sc as plsc

_STRIDE = 8.0
_NC, _NS, _L = 2, 16, 16
_NW = _NC * _NS
_CHUNK = 512


def _sigmoid16(x):
    return 1.0 / (1.0 + jnp.exp(-x))


def _sqrt16(y):
    i = lax.bitcast_convert_type(y, jnp.int32)
    x = lax.bitcast_convert_type((i >> 1) + 0x1FBD1DF5, jnp.float32)
    x = 0.5 * (x + y / x)
    x = 0.5 * (x + y / x)
    return x


def _sc_body(center_hbm, logits_hbm, idxo_hbm, scoreo_hbm,
             lbuf, cbuf, oibuf, osbuf, lsem, *, n_pix, n_cls):
    ppw = n_pix // _NW
    nch = ppw // _CHUNK
    cw = _CHUNK * n_cls
    wid = lax.axis_index("s") * _NC + lax.axis_index("c")
    base = wid * ppw

    def logits_copy(ch, slot):
        return pltpu.make_async_copy(
            logits_hbm.at[pl.ds((base + ch * _CHUNK) * n_cls, cw)],
            lbuf.at[pl.ds(slot * cw, cw)], lsem.at[slot])

    pltpu.sync_copy(center_hbm.at[pl.ds(base, ppw)], cbuf)
    logits_copy(0, 0).start()

    @pl.loop(0, nch)
    def _chunk(ch):
        slot = lax.rem(ch, 2)
        logits_copy(ch, slot).wait()

        @pl.when(ch + 1 < nch)
        def _():
            logits_copy(ch + 1, 1 - slot).start()

        @pl.loop(0, _CHUNK // _L)
        def _group(g):
            p0 = ch * _CHUNK + g * _L
            lane = lax.broadcasted_iota(jnp.int32, (_L,), 0)
            rowbase = slot * cw + (g * _L + lane) * n_cls
            bests, bidxs = [], []
            for k in range(4):
                bests.append(plsc.load_gather(lbuf, [rowbase + k]))
                bidxs.append(jnp.full((_L,), k, jnp.int32))
            for c in range(4, n_cls):
                k = c & 3
                v = plsc.load_gather(lbuf, [rowbase + c])
                gt = v > bests[k]
                bidxs[k] = jnp.where(gt, c, bidxs[k])
                bests[k] = jnp.where(gt, v, bests[k])
            best, bidx = bests[0], bidxs[0]
            for k in range(1, 4):
                gt = bests[k] > best
                bidx = jnp.where(gt, bidxs[k], bidx)
                best = jnp.where(gt, bests[k], best)
            oibuf[pl.ds(p0, _L)] = bidx
            cen = cbuf[pl.ds(p0, _L)]
            osbuf[pl.ds(p0, _L)] = _sqrt16(_sigmoid16(cen) * _sigmoid16(best))

    pltpu.sync_copy(oibuf, idxo_hbm.at[pl.ds(base, ppw)])
    pltpu.sync_copy(osbuf, scoreo_hbm.at[pl.ds(base, ppw)])


def _tc_bbox_body(bbox_ref, out_ref, *, n_h):
    e = jnp.exp(bbox_ref[...]) * _STRIDE
    lane = lax.broadcasted_iota(jnp.int32, e.shape, 1)
    lo_half = (lane & 2) == 0
    partner = jnp.where(lo_half, jnp.roll(e, -2, axis=1),
                        jnp.roll(e, 2, axis=1))
    rvec = (pl.program_id(0) * e.shape[0]
            + lax.broadcasted_iota(jnp.int32, e.shape, 0))
    wx = ((lax.rem(rvec, 4) * 32 + (lane >> 2)).astype(jnp.float32)
          * _STRIDE + _STRIDE * 0.5)
    hy = (lax.rem(rvec >> 2, n_h).astype(jnp.float32)
          * _STRIDE + _STRIDE * 0.5)
    base = jnp.where((lane & 3) == 0, wx, hy)
    out_ref[...] = jnp.where(lo_half, base + (partner - e) * 0.5,
                             partner + e)


def kernel(bbox, center, class_logits, img_h, img_w):
    nB, nH, nW, nCls = class_logits.shape
    n = nB * nH * nW

    center1 = (center * 1.0).reshape(n)
    logits1 = (class_logits * 1.0).reshape(n * nCls)

    mesh = plsc.VectorSubcoreMesh(core_axis_name="c", subcore_axis_name="s")
    sc = pl.kernel(
        functools.partial(_sc_body, n_pix=n, n_cls=nCls),
        mesh=mesh,
        out_type=[
            jax.ShapeDtypeStruct((n,), jnp.int32),
            jax.ShapeDtypeStruct((n,), jnp.float32),
        ],
        scratch_types=[
            pltpu.VMEM((2 * _CHUNK * nCls,), jnp.float32),
            pltpu.VMEM((n // _NW,), jnp.float32),
            pltpu.VMEM((n // _NW,), jnp.int32),
            pltpu.VMEM((n // _NW,), jnp.float32),
            pltpu.SemaphoreType.DMA((2,)),
        ],
        compiler_params=pltpu.CompilerParams(needs_layout_passes=False),
    )
    idx, score = sc(center1, logits1)

    rows = 512
    bbox2 = bbox.reshape(n // 32, 128)
    bbox_out = pl.pallas_call(
        functools.partial(_tc_bbox_body, n_h=nH),
        grid=(n // 32 // rows,),
        in_specs=[pl.BlockSpec((rows, 128), lambda i: (i, 0))],
        out_specs=pl.BlockSpec((rows, 128), lambda i: (i, 0)),
        out_shape=jax.ShapeDtypeStruct((n // 32, 128), jnp.float32),
    )(bbox2)

    return (bbox_out.reshape(nB, nH * nW, 4),
            idx.reshape(nB, nH * nW),
            score.reshape(nB, nH * nW))


# speedup vs baseline: 1.3088x; 1.2960x over previous
import functools

import jax
import jax.numpy as jnp
from jax import lax
from jax.experimental import pallas as pl
from jax.experimental.pallas import tpu as pltpu
from jax.experimental.pallas import tpu_sc as plsc

_STRIDE = 8.0
_NC, _NS, _L = 2, 16, 16
_NW = _NC * _NS
_CHUNK = 512


def _sigmoid16(x):
    return 1.0 / (1.0 + jnp.exp(-x))


def _sqrt16(y):
    i = lax.bitcast_convert_type(y, jnp.int32)
    x = lax.bitcast_convert_type((i >> 1) + 0x1FBD1DF5, jnp.float32)
    x = 0.5 * (x + y / x)
    x = 0.5 * (x + y / x)
    return x


def _sc_body(center_hbm, logits_hbm, idxo_hbm, scoreo_hbm,
             lbuf, cbuf, oibuf, osbuf, lsem, *, n_pix, n_cls, n_h, n_w):
    ppw = n_pix // _NW
    rpw = ppw // n_w
    rch = 2
    nch = rpw // rch
    chpx = rch * n_w
    wid = lax.axis_index("s") * _NC + lax.axis_index("c")
    base = wid * ppw
    rbase = wid * rpw

    def logits_copy(ch, slot):
        r0 = rbase + ch * rch
        return pltpu.make_async_copy(
            logits_hbm.at[pl.ds(r0, rch)],
            lbuf.at[slot], lsem.at[slot])

    pltpu.sync_copy(center_hbm.at[pl.ds(base, ppw)], cbuf)
    logits_copy(0, 0).start()

    @pl.loop(0, nch)
    def _chunk(ch):
        slot = lax.rem(ch, 2)
        logits_copy(ch, slot).wait()

        @pl.when(ch + 1 < nch)
        def _():
            logits_copy(ch + 1, 1 - slot).start()

        @pl.loop(0, chpx // _L)
        def _group(g):
            p0 = g * _L
            lane = lax.broadcasted_iota(jnp.int32, (_L,), 0)
            pv = p0 + lane
            slotv = jnp.full((_L,), slot, jnp.int32)
            rowv = pv >> 7
            wv = pv & 127
            bests, bidxs = [], []
            for k in range(4):
                bests.append(plsc.load_gather(
                    lbuf, [slotv, rowv, wv, jnp.full((_L,), k, jnp.int32)]))
                bidxs.append(jnp.full((_L,), k, jnp.int32))
            for c in range(4, n_cls):
                k = c & 3
                v = plsc.load_gather(
                    lbuf, [slotv, rowv, wv, jnp.full((_L,), c, jnp.int32)])
                gt = v > bests[k]
                bidxs[k] = jnp.where(gt, c, bidxs[k])
                bests[k] = jnp.where(gt, v, bests[k])
            best, bidx = bests[0], bidxs[0]
            for k in range(1, 4):
                gt = bests[k] > best
                bidx = jnp.where(gt, bidxs[k], bidx)
                best = jnp.where(gt, bests[k], best)
            q0 = ch * chpx + p0
            oibuf[pl.ds(q0, _L)] = bidx
            cen = cbuf[pl.ds(q0, _L)]
            osbuf[pl.ds(q0, _L)] = _sqrt16(_sigmoid16(cen) * _sigmoid16(best))

    pltpu.sync_copy(oibuf, idxo_hbm.at[pl.ds(base, ppw)])
    pltpu.sync_copy(osbuf, scoreo_hbm.at[pl.ds(base, ppw)])


def _tc_bbox_body(bbox_ref, out_ref, *, n_h):
    e = jnp.exp(bbox_ref[...]) * _STRIDE
    lane = lax.broadcasted_iota(jnp.int32, e.shape, 1)
    lo_half = (lane & 2) == 0
    partner = jnp.where(lo_half, jnp.roll(e, -2, axis=1),
                        jnp.roll(e, 2, axis=1))
    rvec = (pl.program_id(0) * e.shape[0]
            + lax.broadcasted_iota(jnp.int32, e.shape, 0))
    wx = ((lax.rem(rvec, 4) * 32 + (lane >> 2)).astype(jnp.float32)
          * _STRIDE + _STRIDE * 0.5)
    hy = (lax.rem(rvec >> 2, n_h).astype(jnp.float32)
          * _STRIDE + _STRIDE * 0.5)
    base = jnp.where((lane & 3) == 0, wx, hy)
    out_ref[...] = jnp.where(lo_half, base + (partner - e) * 0.5,
                             partner + e)


def kernel(bbox, center, class_logits, img_h, img_w):
    nB, nH, nW, nCls = class_logits.shape
    n = nB * nH * nW

    center1 = center.reshape(n)

    mesh = plsc.VectorSubcoreMesh(core_axis_name="c", subcore_axis_name="s")
    sc = pl.kernel(
        functools.partial(_sc_body, n_pix=n, n_cls=nCls, n_h=nH, n_w=nW),
        mesh=mesh,
        out_type=[
            jax.ShapeDtypeStruct((n,), jnp.int32),
            jax.ShapeDtypeStruct((n,), jnp.float32),
        ],
        scratch_types=[
            pltpu.VMEM((2, 2, nW, nCls), jnp.float32),
            pltpu.VMEM((n // _NW,), jnp.float32),
            pltpu.VMEM((n // _NW,), jnp.int32),
            pltpu.VMEM((n // _NW,), jnp.float32),
            pltpu.SemaphoreType.DMA((2,)),
        ],
        compiler_params=pltpu.CompilerParams(
            needs_layout_passes=False, use_tc_tiling_on_sc=True),
    )
    idx, score = sc(center1, class_logits.reshape(nB * nH, nW, nCls))

    rows = 512
    bbox2 = bbox.reshape(n // 32, 128)
    bbox_out = pl.pallas_call(
        functools.partial(_tc_bbox_body, n_h=nH),
        grid=(n // 32 // rows,),
        in_specs=[pl.BlockSpec((rows, 128), lambda i: (i, 0))],
        out_specs=pl.BlockSpec((rows, 128), lambda i: (i, 0)),
        out_shape=jax.ShapeDtypeStruct((n // 32, 128), jnp.float32),
    )(bbox2)

    return (bbox_out.reshape(nB, nH * nW, 4),
            idx.reshape(nB, nH * nW),
            score.reshape(nB, nH * nW))
